# baseline (speedup 1.0000x reference)
import functools

import jax
import jax.numpy as jnp
from jax import lax
from jax.experimental import pallas as pl
from jax.experimental.pallas import tpu as pltpu
from jax.experimental.pallas import tpu_sc as plsc

DIM = 64
NC, NS, L = 2, 16, 16
NW = NC * NS
BLK = 128
NBUF = 4


@functools.lru_cache(maxsize=None)
def _build_gather(B, H, vocab):
    nbc = B // BLK
    n_blocks = nbc * H
    per_w = n_blocks // NW
    n_outer = per_w // NBUF
    mesh = plsc.VectorSubcoreMesh(core_axis_name="c", subcore_axis_name="s")

    @functools.partial(
        pl.kernel,
        mesh=mesh,
        out_type=jax.ShapeDtypeStruct((H, 8, nbc, 8, BLK), jnp.float32),
        compiler_params=pltpu.CompilerParams(
            use_tc_tiling_on_sc=False, needs_layout_passes=False),
        scratch_types=[
            pltpu.VMEM((per_w * BLK,), jnp.int32),
            pltpu.VMEM((NBUF, BLK, DIM), jnp.float32),
            pltpu.VMEM((NBUF, 8, 8, BLK), jnp.float32),
            pltpu.SemaphoreType.DMA((NBUF,)),
            pltpu.SemaphoreType.DMA((NBUF,)),
        ],
    )
    def gather_kernel(xq_hbm, tp_hbm, out_hbm, idx_all, rows_v,
                      trans_v, gsem, osem):
        wid = lax.axis_index("s") * NC + lax.axis_index("c")
        iota = lax.iota(jnp.int32, L)
        zero = iota * 0
        pltpu.sync_copy(
            xq_hbm.at[pl.ds(pl.multiple_of(wid * per_w * BLK, BLK),
                            per_w * BLK)],
            idx_all)

        def gather_descr(t, b):
            off = pl.multiple_of(t * BLK, BLK)
            return pltpu.make_async_copy(
                tp_hbm.at[idx_all.at[pl.ds(off, BLK)]], rows_v.at[b],
                gsem.at[b])

        def out_descr(t, b):
            bid = wid * per_w + t
            h = bid // nbc
            bc = lax.rem(bid, nbc)
            return pltpu.make_async_copy(
                trans_v.at[b], out_hbm.at[h, :, bc], osem.at[b])

        def transpose_and_out(t, b):
            rows16s = [iota + cg * L for cg in range(BLK // L)]

            @plsc.parallel_loop(0, DIM)
            def _(d):
                br = lax.div(d, 8)
                r = lax.rem(d, 8)
                dv = zero + d
                for cg in range(BLK // L):
                    vals = plsc.load_gather(
                        rows_v.at[b], [rows16s[cg], dv])
                    trans_v[b, br, r, pl.ds(cg * L, L)] = vals

            out_descr(t, b).start()

        for t0 in range(NBUF - 1):
            gather_descr(t0, t0).start()

        def outer(o, carry):
            for b in range(NBUF):
                t = o * NBUF + b
                nb = (b + NBUF - 1) % NBUF
                if b == 0:
                    gather_descr(t + NBUF - 1, nb).start()
                else:
                    @pl.when(o < n_outer - 1)
                    def _():
                        gather_descr(t + NBUF - 1, nb).start()
                gather_descr(t, b).wait()

                @pl.when(o > 0)
                def _():
                    out_descr(t, b).wait()
                transpose_and_out(t, b)
            return carry

        lax.fori_loop(0, n_outer, outer, 0)
        for b in range(NBUF):
            out_descr(per_w - NBUF + b, b).wait()

    return gather_kernel


def kernel(x, table):
    B, H = x.shape
    vocab = table.shape[0]
    xq = x.T.reshape(B * H).astype(jnp.int32)
    out5 = _build_gather(B, H, vocab)(xq, table)
    return out5.transpose(2, 4, 0, 1, 3).reshape(B, H, DIM)


# speedup vs baseline: 1.6619x; 1.6619x over previous
import functools

import jax
import jax.numpy as jnp
from jax import lax
from jax.experimental import pallas as pl
from jax.experimental.pallas import tpu as pltpu
from jax.experimental.pallas import tpu_sc as plsc

DIM = 64
NC, NS, L = 2, 16, 16
NW = NC * NS
BLK = 128
NBUF = 4


@functools.lru_cache(maxsize=None)
def _build_gather(B, H, vocab):
    nbc = B // BLK
    n_blocks = nbc * H
    per_w = n_blocks // NW
    n_outer = per_w // NBUF
    mesh = plsc.VectorSubcoreMesh(core_axis_name="c", subcore_axis_name="s")

    @functools.partial(
        pl.kernel,
        mesh=mesh,
        out_type=jax.ShapeDtypeStruct((H, 8, nbc, 8, BLK), jnp.float32),
        compiler_params=pltpu.CompilerParams(
            use_tc_tiling_on_sc=False, needs_layout_passes=False),
        scratch_types=[
            pltpu.VMEM((per_w * BLK,), jnp.int32),
            pltpu.VMEM((NBUF, BLK, DIM), jnp.float32),
            pltpu.VMEM((NBUF, DIM, BLK + 1), jnp.float32),
            pltpu.SemaphoreType.DMA((NBUF,)),
            pltpu.SemaphoreType.DMA((NBUF,)),
        ],
    )
    def gather_kernel(xq_hbm, tp_hbm, out_hbm, idx_all, rows_v,
                      trans_v, gsem, osem):
        wid = lax.axis_index("s") * NC + lax.axis_index("c")
        iota = lax.iota(jnp.int32, L)
        zero = iota * 0
        pltpu.sync_copy(
            xq_hbm.at[pl.ds(pl.multiple_of(wid * per_w * BLK, BLK),
                            per_w * BLK)],
            idx_all)

        def gather_descr(t, b):
            off = pl.multiple_of(t * BLK, BLK)
            return pltpu.make_async_copy(
                tp_hbm.at[idx_all.at[pl.ds(off, BLK)]], rows_v.at[b],
                gsem.at[b])

        def out_descrs(t, b):
            bid = wid * per_w + t
            h = bid // nbc
            bc = lax.rem(bid, nbc)
            return [
                pltpu.make_async_copy(
                    trans_v.at[b, pl.ds(br * 8, 8), pl.ds(0, BLK)],
                    out_hbm.at[h, br, bc], osem.at[b])
                for br in range(8)
            ]

        def transpose_and_out(t, b):
            dvecs = [iota + dg * L for dg in range(DIM // L)]

            @plsc.parallel_loop(0, BLK)
            def _(c):
                cv = zero + c
                for dg in range(DIM // L):
                    vals = rows_v[b, c, pl.ds(dg * L, L)]
                    plsc.store_scatter(
                        trans_v.at[b], [dvecs[dg], cv], vals)

            for cp in out_descrs(t, b):
                cp.start()

        for t0 in range(NBUF - 1):
            gather_descr(t0, t0).start()

        def outer(o, carry):
            for b in range(NBUF):
                t = o * NBUF + b
                nb = (b + NBUF - 1) % NBUF
                if b == 0:
                    gather_descr(t + NBUF - 1, nb).start()
                else:
                    @pl.when(o < n_outer - 1)
                    def _():
                        gather_descr(t + NBUF - 1, nb).start()
                gather_descr(t, b).wait()

                @pl.when(o > 0)
                def _():
                    for cp in out_descrs(t, b):
                        cp.wait()
                transpose_and_out(t, b)
            return carry

        lax.fori_loop(0, n_outer, outer, 0)
        for b in range(NBUF):
            for cp in out_descrs(per_w - NBUF + b, b):
                cp.wait()

    return gather_kernel


def kernel(x, table):
    B, H = x.shape
    vocab = table.shape[0]
    xq = x.T.reshape(B * H).astype(jnp.int32)
    out5 = _build_gather(B, H, vocab)(xq, table)
    return out5.transpose(2, 4, 0, 1, 3).reshape(B, H, DIM)
